# baseline (speedup 1.0000x reference)
import functools

import jax
import jax.numpy as jnp
from jax import lax
from jax.experimental import pallas as pl
from jax.experimental.pallas import tpu as pltpu
from jax.experimental.pallas import tpu_sc as plsc

_SQRT_2_OVER_PI = 0.7978845608028654


def _gelu_tanh(x):
    return 0.5 * x * (1.0 + jnp.tanh(_SQRT_2_OVER_PI * (x + 0.044715 * x * x * x)))


def _cp_loss_body(x_ref, w1_ref, b1_ref, w2_ref, b2_ref, bits_ref, thr_ref,
                  tneed_ref, out_ref, eqc_ref):
    i = pl.program_id(0)

    @pl.when(i == 0)
    def _():
        eqc_ref[...] = jnp.zeros_like(eqc_ref)

    bits = bits_ref[...]
    thr = thr_ref[...]
    m_gt = bits > thr
    m_eq = bits == thr
    me = m_eq.astype(jnp.float32)
    bm = bits.shape[0]
    r = lax.broadcasted_iota(jnp.int32, (bm, bm), 0)
    cc = lax.broadcasted_iota(jnp.int32, (bm, bm), 1)
    ltri = (r > cc).astype(jnp.float32)
    pre = lax.dot_general(ltri, me, (((1,), (0,)), ((), ())),
                          preferred_element_type=jnp.float32)
    eqrank = pre + eqc_ref[...]
    eqc_ref[...] += jnp.sum(me, axis=0, keepdims=True)
    tnf = tneed_ref[...].astype(jnp.float32)
    m = jnp.logical_or(m_gt, jnp.logical_and(m_eq, eqrank < tnf))
    m = m.astype(jnp.float32)

    x = x_ref[...]
    h = lax.dot_general(x, w1_ref[...], (((1,), (1,)), ((), ())),
                        preferred_element_type=jnp.float32)
    h = _gelu_tanh(h + b1_ref[...])
    logits = lax.dot_general(h, w2_ref[...],
                             (((1,), (1,)), ((), ())),
                             preferred_element_type=jnp.float32)
    logits = logits + b2_ref[...]
    bce = jnp.maximum(logits, 0.0) - logits * m + jnp.log1p(jnp.exp(-jnp.abs(logits)))
    s = jnp.sum(bce)

    @pl.when(i == 0)
    def _():
        out_ref[...] = jnp.zeros_like(out_ref)

    out_ref[...] += s


def _cp_loss(xf, cp_w1, cp_b1, cp_w2, cp_b2, bits_full, thr_row, tneed_row):
    bs, d = xf.shape
    E = cp_w2.shape[0]
    bm = 256
    grid = (bs // bm,)
    out = pl.pallas_call(
        _cp_loss_body,
        grid=grid,
        in_specs=[
            pl.BlockSpec((bm, d), lambda i: (i, 0)),
            pl.BlockSpec((d, d), lambda i: (0, 0)),
            pl.BlockSpec((1, d), lambda i: (0, 0)),
            pl.BlockSpec((E, d), lambda i: (0, 0)),
            pl.BlockSpec((1, E), lambda i: (0, 0)),
            pl.BlockSpec((bm, E), lambda i: (i, 0)),
            pl.BlockSpec((1, E), lambda i: (0, 0)),
            pl.BlockSpec((1, E), lambda i: (0, 0)),
        ],
        out_specs=pl.BlockSpec((1, 1), lambda i: (0, 0)),
        out_shape=jax.ShapeDtypeStruct((1, 1), jnp.float32),
        scratch_shapes=[pltpu.VMEM((1, E), jnp.float32)],
    )(xf, cp_w1, cp_b1.reshape(1, d),
      cp_w2, cp_b2.reshape(1, E), bits_full, thr_row.reshape(1, E),
      tneed_row.reshape(1, E))
    return out[0, 0] / (bs * E)


def _expert_mlp_body(y_ref, ln_w_ref, ln_b_ref, fc1_ref, b1_ref, fc2_ref,
                     b2_ref, w_ref, z_ref, ln_ref):
    e = pl.program_id(0)
    j = pl.program_id(1)
    nj = pl.num_programs(1)
    ddb = fc1_ref.shape[1]

    @pl.when(j == 0)
    def _():
        yv = y_ref[...]
        mu = jnp.mean(yv, axis=1, keepdims=True)
        var = jnp.mean((yv - mu) ** 2, axis=1, keepdims=True)
        ln = (yv - mu) * lax.rsqrt(var + 1e-5) * ln_w_ref[...] + ln_b_ref[...]
        ln_ref[...] = ln

    ln = ln_ref[...]
    h = lax.dot_general(ln, fc1_ref[0], (((1,), (1,)), ((), ())),
                        preferred_element_type=jnp.float32)
    h = _gelu_tanh(h + b1_ref[pl.ds(e, 1), pl.ds(pl.multiple_of(j * ddb, 128), ddb)])
    zp = lax.dot_general(h, fc2_ref[0],
                         (((1,), (1,)), ((), ())),
                         preferred_element_type=jnp.float32)

    @pl.when(j == 0)
    def _():
        z_ref[...] = jnp.broadcast_to(b2_ref[pl.ds(e, 1), :], z_ref.shape)

    z_ref[...] += zp

    @pl.when(j == nj - 1)
    def _():
        wf = w_ref[...]
        col = lax.broadcasted_iota(jnp.int32, wf.shape, 1) == e
        w = jnp.sum(jnp.where(col, wf, 0.0), axis=1, keepdims=True)
        z_ref[...] = z_ref[...] * w


def _expert_mlp(y, ln_w, ln_b, fc1s, b1s, fc2s, b2s, w_T):
    E, dd, d = fc1s.shape
    k = w_T.shape[1]
    ddb = 2048
    nj = dd // ddb
    grid = (E, nj)
    return pl.pallas_call(
        _expert_mlp_body,
        grid=grid,
        in_specs=[
            pl.BlockSpec((k, d), lambda e, j: (e, 0)),
            pl.BlockSpec((1, d), lambda e, j: (0, 0)),
            pl.BlockSpec((1, d), lambda e, j: (0, 0)),
            pl.BlockSpec((1, ddb, d), lambda e, j: (e, j, 0)),
            pl.BlockSpec((E, dd), lambda e, j: (0, 0)),
            pl.BlockSpec((1, d, ddb), lambda e, j: (e, 0, j)),
            pl.BlockSpec((E, d), lambda e, j: (0, 0)),
            pl.BlockSpec((k, E), lambda e, j: (0, 0)),
        ],
        out_specs=pl.BlockSpec((k, d), lambda e, j: (e, 0)),
        out_shape=jax.ShapeDtypeStruct((E * k, d), jnp.float32),
        scratch_shapes=[pltpu.VMEM((k, d), jnp.float32)],
        compiler_params=pltpu.CompilerParams(
            dimension_semantics=("arbitrary", "arbitrary")),
    )(y.reshape(E * k, d), ln_w.reshape(1, d), ln_b.reshape(1, d),
      fc1s, b1s, fc2s, b2s,
      w_T.T)


def _sc_select_body(scores_hbm, bits_hbm, x_hbm, idx_out, w_out, thr_out,
                    tneed_out, y_out, scores_v, bits_v, idx_g, w_g, idx_e,
                    w_e, tv, gidx_v, rows_v, idx_sh, sem):
    k = 256
    c = lax.axis_index("c")
    s = lax.axis_index("s")

    s4 = jnp.remainder(s, 4)
    e = c * 4 + s4
    pltpu.sync_copy(scores_hbm.at[e], scores_v)
    pltpu.sync_copy(bits_hbm.at[e], bits_v)

    def hsum(v):
        t = v[0]
        for j in range(1, 16):
            t = t + v[j]
        return t

    def count_ge(t):
        def b(i, acc):
            vv = bits_v[pl.ds(i * 16, 16)]
            return acc + jnp.where(vv >= t, 1, 0).astype(jnp.int32)
        acc = lax.fori_loop(0, 128, b, jnp.zeros((16,), jnp.int32), unroll=8)
        return hsum(acc)

    lo = jnp.int32(0)
    hi = jnp.int32(0x3F800001)
    for _ in range(31):
        mid = lo + ((hi - lo) >> 1)
        gei = (count_ge(mid) >= k).astype(jnp.int32)
        lo = gei * mid + (1 - gei) * lo
        hi = gei * hi + (1 - gei) * mid
    thr = lo
    g = count_ge(thr + 1)
    t_need = k - g

    def build(i, carry):
        og, oe = carry
        vv = bits_v[pl.ds(i * 16, 16)]
        sv = scores_v[pl.ds(i * 16, 16)]
        for j in range(16):
            val = vv[j]
            wv = sv[j]
            tok = i * 16 + j
            idx_g[pl.ds(og, 16)] = jnp.broadcast_to(tok, (16,))
            w_g[pl.ds(og, 16)] = jnp.broadcast_to(wv, (16,))
            og = og + (val > thr).astype(jnp.int32)
            idx_e[pl.ds(oe, 16)] = jnp.broadcast_to(tok, (16,))
            w_e[pl.ds(oe, 16)] = jnp.broadcast_to(wv, (16,))
            oe = oe + (val == thr).astype(jnp.int32)
        return og, oe

    g_end, _ = lax.fori_loop(0, 128, build, (jnp.int32(0), jnp.int32(0)))

    for cch in range(16):
        ev = idx_e[pl.ds(cch * 16, 16)]
        wvv = w_e[pl.ds(cch * 16, 16)]
        idx_g[pl.ds(g_end + cch * 16, 16)] = ev
        w_g[pl.ds(g_end + cch * 16, 16)] = wvv

    @pl.when(s < 4)
    def _write():
        pltpu.sync_copy(idx_g.at[pl.ds(0, 256)], idx_out.at[e])
        pltpu.sync_copy(w_g.at[pl.ds(0, 256)], w_out.at[e])
        tv[...] = jnp.broadcast_to(thr, (16,))
        pltpu.sync_copy(tv, thr_out.at[e])
        tv[...] = jnp.broadcast_to(t_need, (16,))
        pltpu.sync_copy(tv, tneed_out.at[e])
        pltpu.sync_copy(idx_g.at[pl.ds(0, 256)], idx_sh.at[s4])

    plsc.subcore_barrier()

    e_l = s // 4
    q = s - e_l * 4
    pltpu.sync_copy(idx_sh.at[e_l, pl.ds(q * 64, 64)], gidx_v)
    pltpu.async_copy(x_hbm.at[gidx_v], rows_v, sem).wait()
    base = (c * 4 + e_l) * 256 + q * 64
    pltpu.sync_copy(rows_v, y_out.at[pl.ds(base, 64)])


def _sc_select_gather(scores_T, xf):
    E, bs = scores_T.shape
    d = xf.shape[1]
    k = bs // E
    mesh = plsc.VectorSubcoreMesh(core_axis_name="c", subcore_axis_name="s")
    fn = functools.partial(
        pl.kernel,
        mesh=mesh,
        out_type=[
            jax.ShapeDtypeStruct((E, k), jnp.int32),
            jax.ShapeDtypeStruct((E, k), jnp.float32),
            jax.ShapeDtypeStruct((E, 16), jnp.int32),
            jax.ShapeDtypeStruct((E, 16), jnp.int32),
            jax.ShapeDtypeStruct((bs, d), jnp.float32),
        ],
        scratch_types=[
            pltpu.VMEM((bs,), jnp.float32),
            pltpu.VMEM((bs,), jnp.int32),
            pltpu.VMEM((528,), jnp.int32),
            pltpu.VMEM((528,), jnp.float32),
            pltpu.VMEM((272,), jnp.int32),
            pltpu.VMEM((272,), jnp.float32),
            pltpu.VMEM((16,), jnp.int32),
            pltpu.VMEM((64,), jnp.int32),
            pltpu.VMEM((64, d), jnp.float32),
            pltpu.VMEM_SHARED((4, k), jnp.int32),
            pltpu.SemaphoreType.DMA,
        ],
    )(_sc_select_body)
    bits_T = lax.bitcast_convert_type(scores_T, jnp.int32)
    return fn(scores_T, bits_T, xf)


def kernel(x, ln_w, ln_b, gate_w, cp_w1, cp_b1, cp_w2, cp_b2, fc1s, b1s, fc2s, b2s):
    og_shape = x.shape
    d = x.shape[-1]
    E = gate_w.shape[0]
    xf = x.reshape(-1, d)
    bs = xf.shape[0]
    k = int(bs * 1.0) // E

    scores = (jnp.tanh(xf @ gate_w.T) + 1.0) / 2.0

    idx_T, w_T, thrv, tneedv, y = _sc_select_gather(scores.T, xf)

    bits_full = lax.bitcast_convert_type(scores, jnp.int32)
    cap_loss = _cp_loss(xf, cp_w1, cp_b1, cp_w2, cp_b2, bits_full,
                        thrv[:, 0], tneedv[:, 0])

    flat_idx = idx_T.reshape(-1)
    z = _expert_mlp(y, ln_w, ln_b, fc1s, b1s, fc2s, b2s, w_T)

    out = xf.at[flat_idx].add(z)
    return out.reshape(og_shape), cap_loss


# speedup vs baseline: 1.0361x; 1.0361x over previous
import functools

import jax
import jax.numpy as jnp
from jax import lax
from jax.experimental import pallas as pl
from jax.experimental.pallas import tpu as pltpu
from jax.experimental.pallas import tpu_sc as plsc

_SQRT_2_OVER_PI = 0.7978845608028654


def _gelu_tanh(x):
    return 0.5 * x * (1.0 + jnp.tanh(_SQRT_2_OVER_PI * (x + 0.044715 * x * x * x)))


def _cp_logits_body(x_ref, w1_ref, b1_ref, w2_ref, b2_ref, out_ref):
    x = x_ref[...]
    h = lax.dot_general(x, w1_ref[...], (((1,), (1,)), ((), ())),
                        preferred_element_type=jnp.float32)
    h = _gelu_tanh(h + b1_ref[...])
    logits = lax.dot_general(h, w2_ref[...],
                             (((1,), (1,)), ((), ())),
                             preferred_element_type=jnp.float32)
    out_ref[...] = logits + b2_ref[...]


def _cp_logits(xf, cp_w1, cp_b1, cp_w2, cp_b2):
    bs, d = xf.shape
    E = cp_w2.shape[0]
    bm = 256
    return pl.pallas_call(
        _cp_logits_body,
        grid=(bs // bm,),
        in_specs=[
            pl.BlockSpec((bm, d), lambda i: (i, 0)),
            pl.BlockSpec((d, d), lambda i: (0, 0)),
            pl.BlockSpec((1, d), lambda i: (0, 0)),
            pl.BlockSpec((E, d), lambda i: (0, 0)),
            pl.BlockSpec((1, E), lambda i: (0, 0)),
        ],
        out_specs=pl.BlockSpec((bm, E), lambda i: (i, 0)),
        out_shape=jax.ShapeDtypeStruct((bs, E), jnp.float32),
    )(xf, cp_w1, cp_b1.reshape(1, d), cp_w2, cp_b2.reshape(1, E))


def _bce_body(logits_ref, bits_ref, thr_ref, tneed_ref, out_ref, eqc_ref):
    i = pl.program_id(0)

    @pl.when(i == 0)
    def _():
        eqc_ref[...] = jnp.zeros_like(eqc_ref)
        out_ref[...] = jnp.zeros_like(out_ref)

    bits = bits_ref[...]
    thr = thr_ref[...]
    m_gt = bits > thr
    m_eq = bits == thr
    me = m_eq.astype(jnp.float32)
    bm = bits.shape[0]
    r = lax.broadcasted_iota(jnp.int32, (bm, bm), 0)
    cc = lax.broadcasted_iota(jnp.int32, (bm, bm), 1)
    ltri = (r > cc).astype(jnp.float32)
    pre = lax.dot_general(ltri, me, (((1,), (0,)), ((), ())),
                          preferred_element_type=jnp.float32)
    eqrank = pre + eqc_ref[...]
    eqc_ref[...] += jnp.sum(me, axis=0, keepdims=True)
    tnf = tneed_ref[...].astype(jnp.float32)
    m = jnp.logical_or(m_gt, jnp.logical_and(m_eq, eqrank < tnf))
    m = m.astype(jnp.float32)

    logits = logits_ref[...]
    bce = jnp.maximum(logits, 0.0) - logits * m + jnp.log1p(jnp.exp(-jnp.abs(logits)))
    out_ref[...] += jnp.sum(bce)


def _cp_loss(logits, bits_full, thr_row, tneed_row):
    bs, E = logits.shape
    bm = 256
    out = pl.pallas_call(
        _bce_body,
        grid=(bs // bm,),
        in_specs=[
            pl.BlockSpec((bm, E), lambda i: (i, 0)),
            pl.BlockSpec((bm, E), lambda i: (i, 0)),
            pl.BlockSpec((1, E), lambda i: (0, 0)),
            pl.BlockSpec((1, E), lambda i: (0, 0)),
        ],
        out_specs=pl.BlockSpec((1, 1), lambda i: (0, 0)),
        out_shape=jax.ShapeDtypeStruct((1, 1), jnp.float32),
        scratch_shapes=[pltpu.VMEM((1, E), jnp.float32)],
    )(logits, bits_full, thr_row.reshape(1, E), tneed_row.reshape(1, E))
    return out[0, 0] / (bs * E)


def _expert_mlp_body(y_ref, ln_w_ref, ln_b_ref, fc1_ref, b1_ref, fc2_ref,
                     b2_ref, w_ref, z_ref, ln_ref):
    e = pl.program_id(0)
    j = pl.program_id(1)
    nj = pl.num_programs(1)
    ddb = fc1_ref.shape[1]

    @pl.when(j == 0)
    def _():
        yv = y_ref[...]
        mu = jnp.mean(yv, axis=1, keepdims=True)
        var = jnp.mean((yv - mu) ** 2, axis=1, keepdims=True)
        ln = (yv - mu) * lax.rsqrt(var + 1e-5) * ln_w_ref[...] + ln_b_ref[...]
        ln_ref[...] = ln

    ln = ln_ref[...]
    h = lax.dot_general(ln, fc1_ref[0], (((1,), (1,)), ((), ())),
                        preferred_element_type=jnp.float32)
    h = _gelu_tanh(h + b1_ref[pl.ds(e, 1), pl.ds(pl.multiple_of(j * ddb, 128), ddb)])
    zp = lax.dot_general(h, fc2_ref[0],
                         (((1,), (1,)), ((), ())),
                         preferred_element_type=jnp.float32)

    @pl.when(j == 0)
    def _():
        z_ref[...] = jnp.broadcast_to(b2_ref[pl.ds(e, 1), :], z_ref.shape)

    z_ref[...] += zp

    @pl.when(j == nj - 1)
    def _():
        wf = w_ref[...]
        col = lax.broadcasted_iota(jnp.int32, wf.shape, 1) == e
        w = jnp.sum(jnp.where(col, wf, 0.0), axis=1, keepdims=True)
        z_ref[...] = z_ref[...] * w


def _expert_mlp(y, ln_w, ln_b, fc1s, b1s, fc2s, b2s, w_T):
    E, dd, d = fc1s.shape
    k = w_T.shape[1]
    ddb = 2048
    nj = dd // ddb
    grid = (E, nj)
    return pl.pallas_call(
        _expert_mlp_body,
        grid=grid,
        in_specs=[
            pl.BlockSpec((k, d), lambda e, j: (e, 0)),
            pl.BlockSpec((1, d), lambda e, j: (0, 0)),
            pl.BlockSpec((1, d), lambda e, j: (0, 0)),
            pl.BlockSpec((1, ddb, d), lambda e, j: (e, j, 0)),
            pl.BlockSpec((E, dd), lambda e, j: (0, 0)),
            pl.BlockSpec((1, d, ddb), lambda e, j: (e, 0, j)),
            pl.BlockSpec((E, d), lambda e, j: (0, 0)),
            pl.BlockSpec((k, E), lambda e, j: (0, 0)),
        ],
        out_specs=pl.BlockSpec((k, d), lambda e, j: (e, 0)),
        out_shape=jax.ShapeDtypeStruct((E * k, d), jnp.float32),
        scratch_shapes=[pltpu.VMEM((k, d), jnp.float32)],
        compiler_params=pltpu.CompilerParams(
            dimension_semantics=("arbitrary", "arbitrary")),
    )(y.reshape(E * k, d), ln_w.reshape(1, d), ln_b.reshape(1, d),
      fc1s, b1s, fc2s, b2s,
      w_T.T)


def _sc_select_body(scores_hbm, bits_hbm, x_hbm, idx_out, w_out, thr_out,
                    tneed_out, y_out, scores_v, bits_v, idx_g, w_g, idx_e,
                    w_e, tv, gidx_v, rows_v, idx_sh, sem):
    k = 256
    c = lax.axis_index("c")
    s = lax.axis_index("s")

    s4 = jnp.remainder(s, 4)
    e = c * 4 + s4
    pltpu.sync_copy(scores_hbm.at[e], scores_v)
    pltpu.sync_copy(bits_hbm.at[e], bits_v)

    def hsum(v):
        t = v[0]
        for j in range(1, 16):
            t = t + v[j]
        return t

    def count_ge(t):
        def b(i, acc):
            vv = bits_v[pl.ds(i * 16, 16)]
            return acc + jnp.where(vv >= t, 1, 0).astype(jnp.int32)
        acc = lax.fori_loop(0, 128, b, jnp.zeros((16,), jnp.int32), unroll=8)
        return hsum(acc)

    lo = jnp.int32(0)
    hi = jnp.int32(0x3F800001)
    for _ in range(31):
        mid = lo + ((hi - lo) >> 1)
        gei = (count_ge(mid) >= k).astype(jnp.int32)
        lo = gei * mid + (1 - gei) * lo
        hi = gei * hi + (1 - gei) * mid
    thr = lo
    g = count_ge(thr + 1)
    t_need = k - g

    def build(i, carry):
        og, oe = carry
        vv = bits_v[pl.ds(i * 16, 16)]
        sv = scores_v[pl.ds(i * 16, 16)]
        for j in range(16):
            val = vv[j]
            wv = sv[j]
            tok = i * 16 + j
            idx_g[pl.ds(og, 16)] = jnp.broadcast_to(tok, (16,))
            w_g[pl.ds(og, 16)] = jnp.broadcast_to(wv, (16,))
            og = og + (val > thr).astype(jnp.int32)
            idx_e[pl.ds(oe, 16)] = jnp.broadcast_to(tok, (16,))
            w_e[pl.ds(oe, 16)] = jnp.broadcast_to(wv, (16,))
            oe = oe + (val == thr).astype(jnp.int32)
        return og, oe

    g_end, _ = lax.fori_loop(0, 128, build, (jnp.int32(0), jnp.int32(0)))

    for cch in range(16):
        ev = idx_e[pl.ds(cch * 16, 16)]
        wvv = w_e[pl.ds(cch * 16, 16)]
        idx_g[pl.ds(g_end + cch * 16, 16)] = ev
        w_g[pl.ds(g_end + cch * 16, 16)] = wvv

    @pl.when(s < 4)
    def _write():
        pltpu.sync_copy(idx_g.at[pl.ds(0, 256)], idx_out.at[e])
        pltpu.sync_copy(w_g.at[pl.ds(0, 256)], w_out.at[e])
        tv[...] = jnp.broadcast_to(thr, (16,))
        pltpu.sync_copy(tv, thr_out.at[e])
        tv[...] = jnp.broadcast_to(t_need, (16,))
        pltpu.sync_copy(tv, tneed_out.at[e])
        pltpu.sync_copy(idx_g.at[pl.ds(0, 256)], idx_sh.at[s4])

    plsc.subcore_barrier()

    e_l = s // 4
    q = s - e_l * 4
    pltpu.sync_copy(idx_sh.at[e_l, pl.ds(q * 64, 64)], gidx_v)
    pltpu.async_copy(x_hbm.at[gidx_v], rows_v, sem).wait()
    base = (c * 4 + e_l) * 256 + q * 64
    pltpu.sync_copy(rows_v, y_out.at[pl.ds(base, 64)])


def _sc_select_gather(scores_T, xf):
    E, bs = scores_T.shape
    d = xf.shape[1]
    k = bs // E
    mesh = plsc.VectorSubcoreMesh(core_axis_name="c", subcore_axis_name="s")
    fn = functools.partial(
        pl.kernel,
        mesh=mesh,
        out_type=[
            jax.ShapeDtypeStruct((E, k), jnp.int32),
            jax.ShapeDtypeStruct((E, k), jnp.float32),
            jax.ShapeDtypeStruct((E, 16), jnp.int32),
            jax.ShapeDtypeStruct((E, 16), jnp.int32),
            jax.ShapeDtypeStruct((bs, d), jnp.float32),
        ],
        scratch_types=[
            pltpu.VMEM((bs,), jnp.float32),
            pltpu.VMEM((bs,), jnp.int32),
            pltpu.VMEM((528,), jnp.int32),
            pltpu.VMEM((528,), jnp.float32),
            pltpu.VMEM((272,), jnp.int32),
            pltpu.VMEM((272,), jnp.float32),
            pltpu.VMEM((16,), jnp.int32),
            pltpu.VMEM((64,), jnp.int32),
            pltpu.VMEM((64, d), jnp.float32),
            pltpu.VMEM_SHARED((4, k), jnp.int32),
            pltpu.SemaphoreType.DMA,
        ],
    )(_sc_select_body)
    bits_T = lax.bitcast_convert_type(scores_T, jnp.int32)
    return fn(scores_T, bits_T, xf)


def kernel(x, ln_w, ln_b, gate_w, cp_w1, cp_b1, cp_w2, cp_b2, fc1s, b1s, fc2s, b2s):
    og_shape = x.shape
    d = x.shape[-1]
    E = gate_w.shape[0]
    xf = x.reshape(-1, d)
    bs = xf.shape[0]
    k = int(bs * 1.0) // E

    scores = (jnp.tanh(xf @ gate_w.T) + 1.0) / 2.0

    logits = _cp_logits(xf, cp_w1, cp_b1, cp_w2, cp_b2)

    idx_T, w_T, thrv, tneedv, y = _sc_select_gather(scores.T, xf)

    bits_full = lax.bitcast_convert_type(scores, jnp.int32)
    cap_loss = _cp_loss(logits, bits_full, thrv[:, 0], tneedv[:, 0])

    flat_idx = idx_T.reshape(-1)
    z = _expert_mlp(y, ln_w, ln_b, fc1s, b1s, fc2s, b2s, w_T)

    out = xf.at[flat_idx].add(z)
    return out.reshape(og_shape), cap_loss
